# Initial kernel scaffold; baseline (speedup 1.0000x reference)
import jax
import jax.numpy as jnp
from jax.experimental import pallas as pl


def kernel(feats, transitions, seq_lens):
    raise NotImplementedError("write your pallas kernel here")


# speedup vs baseline: 3.6726x; 3.6726x over previous
import functools

import jax
import jax.numpy as jnp
from jax.experimental import pallas as pl
from jax.experimental.pallas import tpu as pltpu


def _crf_kernel(feats_ref, trans_ref, seq_ref, out_ref, u_scr, m_scr, *, tb_size):
    tb = pl.program_id(1)
    E = jnp.exp(trans_ref[...])
    sl1 = seq_ref[...] - 1

    def alpha_of(U, M):
        s = jnp.sum(U, axis=1, keepdims=True)
        return M + jnp.log(s)

    def body(s, carry):
        U, M, outv = carry
        gt = tb * tb_size + s
        ft = feats_ref[:, s, :]
        S = jax.lax.dot_general(U, E, (((1,), (1,)), ((), ())),
                                preferred_element_type=jnp.float32)
        maxS = jnp.max(S, axis=1, keepdims=True)
        maxf = jnp.max(ft, axis=1, keepdims=True)
        expf = jnp.exp(ft - maxf)
        Un = S * (expf * (1.0 / maxS))
        Mn = M + maxf + jnp.log(maxS)
        a = alpha_of(Un, Mn)
        outv = jnp.where(sl1 == gt, a, outv)
        return Un, Mn, outv

    @pl.when(tb == 0)
    def _():
        f0 = feats_ref[:, 0, :]
        maxf = jnp.max(f0, axis=1, keepdims=True)
        U0 = jnp.exp(f0 - maxf)
        a0 = alpha_of(U0, maxf)
        out0 = jnp.where(sl1 == 0, a0, jnp.zeros_like(a0))
        U, M, outv = jax.lax.fori_loop(1, tb_size, body, (U0, maxf, out0))
        u_scr[...] = U
        m_scr[...] = M
        out_ref[...] = outv

    @pl.when(tb != 0)
    def _():
        U, M, outv = jax.lax.fori_loop(
            0, tb_size, body, (u_scr[...], m_scr[...], out_ref[...]))
        u_scr[...] = U
        m_scr[...] = M
        out_ref[...] = outv


def kernel(feats, transitions, seq_lens):
    B, T, K = feats.shape
    BB = min(256, B)
    TB = min(64, T)
    assert B % BB == 0 and T % TB == 0
    seq2 = seq_lens.reshape(B, 1).astype(jnp.int32)
    out = pl.pallas_call(
        functools.partial(_crf_kernel, tb_size=TB),
        grid=(B // BB, T // TB),
        in_specs=[
            pl.BlockSpec((BB, TB, K), lambda b, t: (b, t, 0)),
            pl.BlockSpec((K, K), lambda b, t: (0, 0)),
            pl.BlockSpec((BB, 1), lambda b, t: (b, 0)),
        ],
        out_specs=pl.BlockSpec((BB, 1), lambda b, t: (b, 0)),
        out_shape=jax.ShapeDtypeStruct((B, 1), jnp.float32),
        scratch_shapes=[
            pltpu.VMEM((BB, K), jnp.float32),
            pltpu.VMEM((BB, 1), jnp.float32),
        ],
        compiler_params=pltpu.CompilerParams(
            dimension_semantics=("parallel", "arbitrary"),
        ),
        name="crf_forward",
    )(feats, transitions, seq2)
    return out


# speedup vs baseline: 16.9903x; 4.6263x over previous
import functools

import jax
import jax.numpy as jnp
from jax.experimental import pallas as pl
from jax.experimental.pallas import tpu as pltpu


def _crf_kernel(feats_ref, trans_ref, seq_ref, out_ref, u_scr, m_scr, *, tb_size):
    tb = pl.program_id(1)
    E = jnp.exp(trans_ref[...])
    sl1 = seq_ref[...] - 1

    def body(s, carry):
        U, M, outv = carry
        gt = tb * tb_size + s
        ft = feats_ref[s]
        S = jnp.dot(E, U, preferred_element_type=jnp.float32)
        P = S * jnp.exp(ft)
        sumP = jnp.sum(P, axis=0, keepdims=True)
        Un = P * (1.0 / sumP)
        Mn = M + jnp.log(sumP)
        outv = jnp.where(sl1 == gt, Mn, outv)
        return Un, Mn, outv

    @pl.when(tb == 0)
    def _():
        P0 = jnp.exp(feats_ref[0])
        s0 = jnp.sum(P0, axis=0, keepdims=True)
        U0 = P0 * (1.0 / s0)
        M0 = jnp.log(s0)
        out0 = jnp.where(sl1 == 0, M0, jnp.zeros_like(M0))
        U, M, outv = jax.lax.fori_loop(1, tb_size, body, (U0, M0, out0),
                                       unroll=4)
        u_scr[...] = U
        m_scr[...] = M
        out_ref[...] = outv

    @pl.when(tb != 0)
    def _():
        U, M, outv = jax.lax.fori_loop(
            0, tb_size, body, (u_scr[...], m_scr[...], out_ref[...]),
            unroll=4)
        u_scr[...] = U
        m_scr[...] = M
        out_ref[...] = outv


def kernel(feats, transitions, seq_lens):
    B, T, K = feats.shape
    BBL = min(512, B)
    TB = min(64, T)
    assert B % BBL == 0 and T % TB == 0
    fT = jnp.transpose(feats, (1, 2, 0))
    seq2 = seq_lens.reshape(1, B).astype(jnp.int32)
    out = pl.pallas_call(
        functools.partial(_crf_kernel, tb_size=TB),
        grid=(B // BBL, T // TB),
        in_specs=[
            pl.BlockSpec((TB, K, BBL), lambda b, t: (t, 0, b)),
            pl.BlockSpec((K, K), lambda b, t: (0, 0)),
            pl.BlockSpec((1, BBL), lambda b, t: (0, b)),
        ],
        out_specs=pl.BlockSpec((1, BBL), lambda b, t: (0, b)),
        out_shape=jax.ShapeDtypeStruct((1, B), jnp.float32),
        scratch_shapes=[
            pltpu.VMEM((K, BBL), jnp.float32),
            pltpu.VMEM((1, BBL), jnp.float32),
        ],
        compiler_params=pltpu.CompilerParams(
            dimension_semantics=("parallel", "arbitrary"),
        ),
        name="crf_forward",
    )(fT, transitions, seq2)
    return out.reshape(B, 1)
